# baseline (speedup 1.0000x reference)
import functools

import jax
import jax.numpy as jnp
from jax import lax
from jax.experimental import pallas as pl
from jax.experimental.pallas import tpu as pltpu
from jax.experimental.pallas import tpu_sc as plsc

_B = 16384
_D = 32
_NC = 2
_NS = 16
_NW = _NC * _NS
_BPW = _B // _NW
_L = 16
_CHUNK = 128
_NCHUNK = _BPW // _CHUNK
_GROUPS = _BPW // _L


def _body(uid_hbm, iid_hbm, uf_hbm, if_hbm, out_hbm,
          uid_v, iid_v, urows, irows, sflat, out_v, sem_u, sem_i):
    wid = lax.axis_index("s") * _NC + lax.axis_index("c")
    base = pl.multiple_of(wid * _BPW, _BPW)

    for c in range(_NCHUNK):
        pltpu.sync_copy(uid_hbm.at[pl.ds(base + c * _CHUNK, _CHUNK)],
                        uid_v.at[c])
        pltpu.sync_copy(iid_hbm.at[pl.ds(base + c * _CHUNK, _CHUNK)],
                        iid_v.at[c])

    copies = []
    for c in range(_NCHUNK):
        dst = pl.ds(c * _CHUNK, _CHUNK)
        copies.append(pltpu.async_copy(uf_hbm.at[uid_v.at[c]],
                                       urows.at[dst], sem_u))
        copies.append(pltpu.async_copy(if_hbm.at[iid_v.at[c]],
                                       irows.at[dst], sem_i))
    for cp in copies:
        cp.wait()

    def fold(b, carry):
        u0 = urows[b, pl.ds(0, _L)]
        u1 = urows[b, pl.ds(_L, _L)]
        v0 = irows[b, pl.ds(0, _L)]
        v1 = irows[b, pl.ds(_L, _L)]
        sflat[pl.ds(pl.multiple_of(b * _L, _L), _L)] = u0 * v0 + u1 * v1
        return carry

    lax.fori_loop(0, _BPW, fold, 0)

    iota16 = lax.iota(jnp.int32, _L) * _L

    def group(g, carry):
        gbase = g * (_L * _L)
        accs = [jnp.zeros((_L,), jnp.float32) for _ in range(4)]
        for d in range(_L):
            idx = iota16 + (gbase + d)
            accs[d % 4] = accs[d % 4] + plsc.load_gather(sflat, [idx])
        res = (accs[0] + accs[1]) + (accs[2] + accs[3])
        out_v[pl.ds(pl.multiple_of(g * _L, _L), _L)] = res
        return carry

    lax.fori_loop(0, _GROUPS, group, 0)

    pltpu.sync_copy(out_v, out_hbm.at[pl.ds(base, _BPW)])


def kernel(user_ids, item_ids, user_factors, item_factors):
    mesh = plsc.VectorSubcoreMesh(core_axis_name="c", subcore_axis_name="s")
    k = functools.partial(
        pl.kernel,
        mesh=mesh,
        out_type=jax.ShapeDtypeStruct((_B,), jnp.float32),
        compiler_params=pltpu.CompilerParams(
            needs_layout_passes=False, use_tc_tiling_on_sc=False),
        scratch_types=[
            pltpu.VMEM((_NCHUNK, _CHUNK), jnp.int32),
            pltpu.VMEM((_NCHUNK, _CHUNK), jnp.int32),
            pltpu.VMEM((_BPW, _D), jnp.float32),
            pltpu.VMEM((_BPW, _D), jnp.float32),
            pltpu.VMEM((_BPW * _L,), jnp.float32),
            pltpu.VMEM((_BPW,), jnp.float32),
            pltpu.SemaphoreType.DMA,
            pltpu.SemaphoreType.DMA,
        ],
    )(_body)
    return k(user_ids.astype(jnp.int32), item_ids.astype(jnp.int32),
             user_factors, item_factors)
